# baseline (speedup 1.0000x reference)
import functools

import jax
import jax.numpy as jnp
from jax import lax
from jax.experimental import pallas as pl
from jax.experimental.pallas import tpu as pltpu
from jax.experimental.pallas import tpu_sc as plsc

K_NB = 128
D = 64
N_KEYS = 1_000_000
N_TILES = 32
LANES = 16
N_CHUNKS = 2

N_T = ((N_KEYS + N_TILES - 1) // N_TILES + 127) // 128 * 128
CH_T = N_T // N_CHUNKS
CH = CH_T * N_TILES
N_PAD = CH * N_CHUNKS
NV = CH_T // LANES
UN = 4

B1 = 100352
G1C = CH // B1
assert CH % B1 == 0 and CH_T % LANES == 0 and NV % UN == 0


def _dist_kernel_for(chunk):
    goff = chunk * CH

    def _dist_kernel(q_ref, k_ref, o_ref):
        qv = q_ref[...]
        kb = k_ref[...]
        kq = lax.dot_general(qv, kb, (((1,), (0,)), ((), ())))
        ksq = jnp.sum(kb * kb, axis=0, keepdims=True)
        qsq = jnp.sum(qv * qv)
        d2 = (qsq + ksq) - 2.0 * kq
        d2 = jnp.maximum(d2, 0.0)
        dist = jnp.sqrt(d2 + 1e-12)
        bits = pltpu.bitcast(dist, jnp.int32)
        i = pl.program_id(0)
        gid = goff + i * B1 + lax.broadcasted_iota(jnp.int32, (1, B1), 1)
        o_ref[...] = jnp.where(gid < N_KEYS, bits,
                               jnp.int32(0x7F800000))[None]

    return _dist_kernel


def _make_dist_call(chunk):
    return pl.pallas_call(
        _dist_kernel_for(chunk),
        grid=(G1C,),
        in_specs=[
            pl.BlockSpec((1, D), lambda i: (0, 0)),
            pl.BlockSpec((D, B1), lambda i, c=chunk: (0, i + c * G1C)),
        ],
        out_specs=pl.BlockSpec((1, 1, B1), lambda i: (i, 0, 0)),
        out_shape=jax.ShapeDtypeStruct((G1C, 1, B1), jnp.int32),
    )


_DIST_CALLS = [_make_dist_call(c) for c in range(N_CHUNKS)]


def _select_body_for(chunk):
    goff = chunk * CH

    def _select_body(dists, out_v, out_i, vals, hist, racc, ltv, lti, eqi,
                     ov, oi):
        c = lax.axis_index("c")
        s = lax.axis_index("s")
        wid = s * 2 + c
        base = wid * CH_T
        pltpu.sync_copy(dists.at[pl.ds(base, CH_T)], vals)

        lane = lax.iota(jnp.int32, LANES)
        ones = jnp.ones((LANES,), jnp.int32)
        zeros16 = jnp.zeros((LANES,), jnp.int32)

        prefix = jnp.int32(0)
        k_rem = jnp.int32(K_NB)
        for r in range(4):
            sh_d = 24 - 8 * r

            def zbody(j):
                hist[pl.ds(j * LANES, LANES)] = zeros16

            plsc.parallel_loop(0, 256, 1, unroll=8)(zbody)

            if r == 0:
                def hbody(i):
                    v = vals[pl.ds(i * LANES, LANES)]
                    byte = (v >> sh_d) & 255
                    plsc.addupdate_scatter(hist, [byte * LANES + lane], ones)
            else:
                sh_hi = 32 - 8 * r
                pref_hi = prefix >> sh_hi

                def hbody(i):
                    v = vals[pl.ds(i * LANES, LANES)]
                    byte = (v >> sh_d) & 255
                    ok = (v >> sh_hi) == pref_hi
                    plsc.addupdate_scatter(hist, [byte * LANES + lane], ones,
                                           mask=ok)

            plsc.parallel_loop(0, NV, 1, unroll=UN)(hbody)

            def cbody(jo, acc):
                for u in range(4):
                    acc = acc + hist[pl.ds((jo * 4 + u) * LANES, LANES)]
                    racc[pl.ds((jo * 4 + u) * LANES, LANES)] = acc
                return acc

            lax.fori_loop(0, 256 // 4, cbody, zeros16)

            def sbody(_, lohi):
                lo, hi = lohi
                mid = (lo + hi) // 2
                sm = jnp.sum(racc[pl.ds(mid * LANES, LANES)])
                ok = sm >= k_rem
                return jnp.where(ok, lo, mid + 1), jnp.where(ok, mid, hi)

            digit, _ = lax.fori_loop(0, 8, sbody,
                                     (jnp.int32(0), jnp.int32(255)))
            pm1 = jnp.maximum(digit - 1, 0)
            cum_before = jnp.where(
                digit > 0, jnp.sum(racc[pl.ds(pm1 * LANES, LANES)]),
                jnp.int32(0))
            k_rem = k_rem - cum_before
            prefix = prefix | (digit << sh_d)

        v_bits = prefix

        def pbody(io, ptrs):
            p_lt, p_eq = ptrs
            for u in range(UN):
                i = io * UN + u
                v = vals[pl.ds(i * LANES, LANES)]
                lt = v < v_bits
                eq = v == v_bits
                gidx = goff + base + i * LANES + lane
                plsc.store_compressed(ltv.at[pl.ds(p_lt, LANES)], v, mask=lt)
                plsc.store_compressed(lti.at[pl.ds(p_lt, LANES)], gidx,
                                      mask=lt)
                plsc.store_compressed(eqi.at[pl.ds(p_eq, LANES)], gidx,
                                      mask=eq)
                p_lt = p_lt + plsc.all_reduce_population_count(lt)[0]
                p_eq = p_eq + plsc.all_reduce_population_count(eq)[0]
            return (p_lt, p_eq)

        lax.fori_loop(0, NV // UN, pbody, (jnp.int32(0), jnp.int32(0)))

        count_lt = jnp.int32(K_NB) - k_rem
        vfull = jnp.broadcast_to(v_bits, (LANES,))
        for j in range(K_NB // LANES):
            pos = j * LANES + lane
            sel = pos < count_lt
            lv = ltv[pl.ds(j * LANES, LANES)]
            li = lti[pl.ds(j * LANES, LANES)]
            ei = plsc.load_gather(eqi, [jnp.maximum(pos - count_lt, 0)])
            ov[pl.ds(j * LANES, LANES)] = jnp.where(sel, lv, vfull)
            oi[pl.ds(j * LANES, LANES)] = jnp.where(sel, li, ei)

        pltpu.sync_copy(ov, out_v.at[wid])
        pltpu.sync_copy(oi, out_i.at[wid])

    return _select_body


@functools.lru_cache(maxsize=N_CHUNKS)
def _make_select_call(chunk):
    return pl.kernel(
        _select_body_for(chunk),
        out_type=(
            jax.ShapeDtypeStruct((N_TILES, K_NB), jnp.int32),
            jax.ShapeDtypeStruct((N_TILES, K_NB), jnp.int32),
        ),
        mesh=plsc.VectorSubcoreMesh(
            core_axis_name="c", subcore_axis_name="s",
            num_cores=2, num_subcores=16),
        compiler_params=pltpu.CompilerParams(needs_layout_passes=False),
        scratch_types=[
            pltpu.VMEM((CH_T,), jnp.int32),
            pltpu.VMEM((256 * LANES,), jnp.int32),
            pltpu.VMEM((256 * LANES,), jnp.int32),
            pltpu.VMEM((K_NB + LANES,), jnp.int32),
            pltpu.VMEM((K_NB + LANES,), jnp.int32),
            pltpu.VMEM((CH_T + LANES,), jnp.int32),
            pltpu.VMEM((K_NB,), jnp.int32),
            pltpu.VMEM((K_NB,), jnp.int32),
        ],
    )


def _merge_kernel(cv_ref, ci_ref, ov_ref, oi_ref):
    lane = lax.broadcasted_iota(jnp.int32, (1, K_NB), 1)
    cand = pltpu.bitcast(cv_ref[...], jnp.float32)

    def body(j, carry):
        vals, oval, oidx = carry
        m = jnp.min(vals)
        sel = vals == m
        i = jnp.min(jnp.where(sel, ci_ref[...], jnp.int32(2**31 - 1)))
        vals = jnp.where(sel & (ci_ref[...] == i), jnp.float32(jnp.inf), vals)
        oval = jnp.where(lane == j, m, oval)
        oidx = jnp.where(lane == j, i, oidx)
        return vals, oval, oidx

    _, oval, oidx = lax.fori_loop(
        0, K_NB, body,
        (cand, jnp.zeros((1, K_NB), jnp.float32),
         jnp.zeros((1, K_NB), jnp.int32)))
    ov_ref[...] = oval
    oi_ref[...] = oidx


_merge_call = pl.pallas_call(
    _merge_kernel,
    out_shape=(
        jax.ShapeDtypeStruct((1, K_NB), jnp.float32),
        jax.ShapeDtypeStruct((1, K_NB), jnp.int32),
    ),
)


def kernel(queries, keys):
    kt = keys.T
    cvs, cis = [], []
    for c in range(N_CHUNKS):
        dists = _DIST_CALLS[c](queries, kt).reshape(-1)
        cv, ci = _make_select_call(c)(dists)
        cvs.append(cv)
        cis.append(ci)
    return _merge_call(jnp.concatenate(cvs, axis=0),
                       jnp.concatenate(cis, axis=0))


# speedup vs baseline: 1.0496x; 1.0496x over previous
import functools

import jax
import jax.numpy as jnp
from jax import lax
from jax.experimental import pallas as pl
from jax.experimental.pallas import tpu as pltpu
from jax.experimental.pallas import tpu_sc as plsc

K_NB = 128
D = 64
N_KEYS = 1_000_000
N_TILES = 32
LANES = 16
N_CHUNKS = 2

N_T = ((N_KEYS + N_TILES - 1) // N_TILES + 127) // 128 * 128
CH_T = N_T // N_CHUNKS
CH = CH_T * N_TILES
N_PAD = CH * N_CHUNKS
NV = CH_T // LANES
UN = 4

B1 = 50176
G1C = CH // B1
assert CH % B1 == 0 and CH_T % LANES == 0 and NV % UN == 0


def _dist_kernel_for(chunk):
    goff = chunk * CH

    def _dist_kernel(q_ref, k_ref, o_ref):
        qv = q_ref[...]
        kb = k_ref[...]
        kq = lax.dot_general(qv, kb, (((1,), (0,)), ((), ())))
        ksq = jnp.sum(kb * kb, axis=0, keepdims=True)
        qsq = jnp.sum(qv * qv)
        d2 = (qsq + ksq) - 2.0 * kq
        d2 = jnp.maximum(d2, 0.0)
        dist = jnp.sqrt(d2 + 1e-12)
        bits = pltpu.bitcast(dist, jnp.int32)
        i = pl.program_id(0)
        gid = goff + i * B1 + lax.broadcasted_iota(jnp.int32, (1, B1), 1)
        o_ref[...] = jnp.where(gid < N_KEYS, bits,
                               jnp.int32(0x7F800000))[None]

    return _dist_kernel


def _make_dist_call(chunk):
    return pl.pallas_call(
        _dist_kernel_for(chunk),
        grid=(G1C,),
        in_specs=[
            pl.BlockSpec((1, D), lambda i: (0, 0)),
            pl.BlockSpec((D, B1), lambda i, c=chunk: (0, i + c * G1C)),
        ],
        out_specs=pl.BlockSpec((1, 1, B1), lambda i: (i, 0, 0)),
        out_shape=jax.ShapeDtypeStruct((G1C, 1, B1), jnp.int32),
    )


_DIST_CALLS = [_make_dist_call(c) for c in range(N_CHUNKS)]


def _select_body_for(chunk):
    goff = chunk * CH

    def _select_body(dists, out_v, out_i, vals, hist, racc, ltv, lti, eqi,
                     ov, oi):
        c = lax.axis_index("c")
        s = lax.axis_index("s")
        wid = s * 2 + c
        base = wid * CH_T
        pltpu.sync_copy(dists.at[pl.ds(base, CH_T)], vals)

        lane = lax.iota(jnp.int32, LANES)
        ones = jnp.ones((LANES,), jnp.int32)
        zeros16 = jnp.zeros((LANES,), jnp.int32)

        prefix = jnp.int32(0)
        k_rem = jnp.int32(K_NB)
        for r in range(4):
            sh_d = 24 - 8 * r

            def zbody(j):
                hist[pl.ds(j * LANES, LANES)] = zeros16

            plsc.parallel_loop(0, 256, 1, unroll=8)(zbody)

            if r == 0:
                def hbody(i):
                    v = vals[pl.ds(i * LANES, LANES)]
                    byte = (v >> sh_d) & 255
                    plsc.addupdate_scatter(hist, [byte * LANES + lane], ones)
            else:
                sh_hi = 32 - 8 * r
                pref_hi = prefix >> sh_hi

                def hbody(i):
                    v = vals[pl.ds(i * LANES, LANES)]
                    byte = (v >> sh_d) & 255
                    ok = (v >> sh_hi) == pref_hi
                    plsc.addupdate_scatter(hist, [byte * LANES + lane], ones,
                                           mask=ok)

            plsc.parallel_loop(0, NV, 1, unroll=UN)(hbody)

            def cbody(jo, acc):
                for u in range(4):
                    acc = acc + hist[pl.ds((jo * 4 + u) * LANES, LANES)]
                    racc[pl.ds((jo * 4 + u) * LANES, LANES)] = acc
                return acc

            lax.fori_loop(0, 256 // 4, cbody, zeros16)

            def sbody(_, lohi):
                lo, hi = lohi
                mid = (lo + hi) // 2
                sm = jnp.sum(racc[pl.ds(mid * LANES, LANES)])
                ok = sm >= k_rem
                return jnp.where(ok, lo, mid + 1), jnp.where(ok, mid, hi)

            digit, _ = lax.fori_loop(0, 8, sbody,
                                     (jnp.int32(0), jnp.int32(255)))
            pm1 = jnp.maximum(digit - 1, 0)
            cum_before = jnp.where(
                digit > 0, jnp.sum(racc[pl.ds(pm1 * LANES, LANES)]),
                jnp.int32(0))
            k_rem = k_rem - cum_before
            prefix = prefix | (digit << sh_d)

        v_bits = prefix

        def pbody(io, ptrs):
            p_lt, p_eq = ptrs
            for u in range(UN):
                i = io * UN + u
                v = vals[pl.ds(i * LANES, LANES)]
                lt = v < v_bits
                eq = v == v_bits
                gidx = goff + base + i * LANES + lane
                plsc.store_compressed(ltv.at[pl.ds(p_lt, LANES)], v, mask=lt)
                plsc.store_compressed(lti.at[pl.ds(p_lt, LANES)], gidx,
                                      mask=lt)
                plsc.store_compressed(eqi.at[pl.ds(p_eq, LANES)], gidx,
                                      mask=eq)
                p_lt = p_lt + plsc.all_reduce_population_count(lt)[0]
                p_eq = p_eq + plsc.all_reduce_population_count(eq)[0]
            return (p_lt, p_eq)

        lax.fori_loop(0, NV // UN, pbody, (jnp.int32(0), jnp.int32(0)))

        count_lt = jnp.int32(K_NB) - k_rem
        vfull = jnp.broadcast_to(v_bits, (LANES,))
        for j in range(K_NB // LANES):
            pos = j * LANES + lane
            sel = pos < count_lt
            lv = ltv[pl.ds(j * LANES, LANES)]
            li = lti[pl.ds(j * LANES, LANES)]
            ei = plsc.load_gather(eqi, [jnp.maximum(pos - count_lt, 0)])
            ov[pl.ds(j * LANES, LANES)] = jnp.where(sel, lv, vfull)
            oi[pl.ds(j * LANES, LANES)] = jnp.where(sel, li, ei)

        pltpu.sync_copy(ov, out_v.at[wid])
        pltpu.sync_copy(oi, out_i.at[wid])

    return _select_body


@functools.lru_cache(maxsize=N_CHUNKS)
def _make_select_call(chunk):
    return pl.kernel(
        _select_body_for(chunk),
        out_type=(
            jax.ShapeDtypeStruct((N_TILES, K_NB), jnp.int32),
            jax.ShapeDtypeStruct((N_TILES, K_NB), jnp.int32),
        ),
        mesh=plsc.VectorSubcoreMesh(
            core_axis_name="c", subcore_axis_name="s",
            num_cores=2, num_subcores=16),
        compiler_params=pltpu.CompilerParams(needs_layout_passes=False),
        scratch_types=[
            pltpu.VMEM((CH_T,), jnp.int32),
            pltpu.VMEM((256 * LANES,), jnp.int32),
            pltpu.VMEM((256 * LANES,), jnp.int32),
            pltpu.VMEM((K_NB + LANES,), jnp.int32),
            pltpu.VMEM((K_NB + LANES,), jnp.int32),
            pltpu.VMEM((CH_T + LANES,), jnp.int32),
            pltpu.VMEM((K_NB,), jnp.int32),
            pltpu.VMEM((K_NB,), jnp.int32),
        ],
    )


def _merge_kernel(cv_ref, ci_ref, ov_ref, oi_ref):
    lane = lax.broadcasted_iota(jnp.int32, (1, K_NB), 1)
    cand = pltpu.bitcast(cv_ref[...], jnp.float32)

    def body(j, carry):
        vals, oval, oidx = carry
        m = jnp.min(vals)
        sel = vals == m
        i = jnp.min(jnp.where(sel, ci_ref[...], jnp.int32(2**31 - 1)))
        vals = jnp.where(sel & (ci_ref[...] == i), jnp.float32(jnp.inf), vals)
        oval = jnp.where(lane == j, m, oval)
        oidx = jnp.where(lane == j, i, oidx)
        return vals, oval, oidx

    _, oval, oidx = lax.fori_loop(
        0, K_NB, body,
        (cand, jnp.zeros((1, K_NB), jnp.float32),
         jnp.zeros((1, K_NB), jnp.int32)))
    ov_ref[...] = oval
    oi_ref[...] = oidx


_merge_call = pl.pallas_call(
    _merge_kernel,
    out_shape=(
        jax.ShapeDtypeStruct((1, K_NB), jnp.float32),
        jax.ShapeDtypeStruct((1, K_NB), jnp.int32),
    ),
)


def kernel(queries, keys):
    kt = keys.T
    cvs, cis = [], []
    for c in range(N_CHUNKS):
        dists = _DIST_CALLS[c](queries, kt).reshape(-1)
        cv, ci = _make_select_call(c)(dists)
        cvs.append(cv)
        cis.append(ci)
    return _merge_call(jnp.concatenate(cvs, axis=0),
                       jnp.concatenate(cis, axis=0))
